# baseline (speedup 1.0000x reference)
import jax
import jax.numpy as jnp
from jax import lax
from jax.experimental import pallas as pl
from jax.experimental.pallas import tpu as pltpu
from jax.experimental.pallas import tpu_sc as plsc

N_NODES = 10000
N_EDGES = 320000
D = 128

NC = 2
NS = 16
NW = NC * NS
EPW = N_EDGES // NW
C = 80
NCH = EPW // C
NP = 10112
N_PER_TILE = NP // NS


def _preproj_body(x_ref, w1_ref, b1_ref, out_ref):
    xa = jnp.dot(x_ref[...], w1_ref[:D, :], preferred_element_type=jnp.float32)
    xb = jnp.dot(x_ref[...], w1_ref[D:, :], preferred_element_type=jnp.float32)
    out_ref[0] = xa.astype(jnp.bfloat16)
    out_ref[1] = (xb + b1_ref[...]).astype(jnp.bfloat16)


def _preproj(x, W1, b1):
    BN = 2000
    grid = (N_NODES // BN,)
    return pl.pallas_call(
        _preproj_body,
        grid=grid,
        in_specs=[
            pl.BlockSpec((BN, D), lambda i: (i, 0)),
            pl.BlockSpec((2 * D, D), lambda i: (0, 0)),
            pl.BlockSpec((1, D), lambda i: (0, 0)),
        ],
        out_specs=pl.BlockSpec((2, BN, D), lambda i: (0, i, 0)),
        out_shape=jax.ShapeDtypeStruct((2, N_NODES, D), jnp.bfloat16),
    )(x, W1, b1.reshape(1, D))


NBUF = 4
DW = D // 2


def _gather_body(tab_hbm, ridx_hbm, cidx_hbm, out_hbm,
                 ia_v, ib_v, ba, bb, sga, sgb, ssa):
    wid = lax.axis_index("s") * NC + lax.axis_index("c")
    e0 = wid * EPW

    pltpu.sync_copy(ridx_hbm.at[pl.ds(e0, EPW)], ia_v)
    pltpu.sync_copy(cidx_hbm.at[pl.ds(e0, EPW)], ib_v)

    def issue_gather(ci, b):
        pltpu.async_copy(tab_hbm.at[ia_v.at[pl.ds(ci * C, C)]], ba[b], sga[b])
        pltpu.async_copy(tab_hbm.at[ib_v.at[pl.ds(ci * C, C)]], bb[b], sgb[b])

    def wait_gather(b):
        pltpu.make_async_copy(tab_hbm.at[pl.ds(0, C)], ba[b], sga[b]).wait()
        pltpu.make_async_copy(tab_hbm.at[pl.ds(0, C)], bb[b], sgb[b]).wait()

    def add_rows(b):
        A, B = ba[b], bb[b]

        def row_add(i, c2):
            for j in range(DW // 16):
                sl = pl.ds(j * 16, 16)
                s = (plsc.bitcast(A[i, sl], jnp.bfloat16)
                     + plsc.bitcast(B[i, sl], jnp.bfloat16))
                A[i, sl] = plsc.bitcast(s, jnp.int32)
            return c2

        lax.fori_loop(0, C, row_add, 0, unroll=4)

    def issue_store(ci, b):
        base = e0 + ci * C
        pltpu.async_copy(ba[b], out_hbm.at[pl.ds(base, C)], ssa[b])

    def drain_store(b):
        pltpu.make_async_copy(ba[b], out_hbm.at[pl.ds(0, C)], ssa[b]).wait()

    for b in range(NBUF):
        issue_store(b, b)

    def body(k, carry):
        g = k * NBUF
        for b in range(NBUF):
            drain_store(b)
            issue_gather(g + b, b)
        for b in range(NBUF):
            wait_gather(b)
            add_rows(b)
            issue_store(g + b, b)
        return carry

    lax.fori_loop(0, (NCH - 1) // NBUF, body, 0)
    drain_store(0)
    issue_gather(NCH - 1, 0)
    wait_gather(0)
    add_rows(0)
    base = e0 + (NCH - 1) * C
    pltpu.sync_copy(ba[0], out_hbm.at[pl.ds(base, C)])
    for b in range(1, NBUF):
        drain_store(b)


def _gather_pairs(xab, ridx, cidxp):
    mesh = plsc.VectorSubcoreMesh(core_axis_name="c", subcore_axis_name="s")
    f = pl.kernel(
        lambda tab, ri, ci, out, iav, ibv, *rest: _gather_body(
            tab, ri, ci, out, iav, ibv,
            rest[0:NBUF], rest[NBUF:2 * NBUF],
            rest[2 * NBUF:3 * NBUF], rest[3 * NBUF:4 * NBUF],
            rest[4 * NBUF:5 * NBUF]),
        out_type=jax.ShapeDtypeStruct((N_EDGES, DW), jnp.int32),
        mesh=mesh,
        compiler_params=pltpu.CompilerParams(use_tc_tiling_on_sc=False,
                                             needs_layout_passes=False),
        scratch_types=(
            [pltpu.VMEM((EPW,), jnp.int32)] * 2
            + [pltpu.VMEM((C, DW), jnp.int32)] * (2 * NBUF)
            + [pltpu.SemaphoreType.DMA] * (3 * NBUF)
        ),
    )
    return f(xab, ridx, cidxp)


def _softplus(h):
    return jnp.maximum(h, 0.0) + jnp.log(1.0 + jnp.exp(-jnp.abs(h)))


def _mlp_body(h_ref, w2_ref, b2_ref, w3_ref, b3_ref, out_ref):
    h = _softplus(h_ref[...].astype(jnp.float32))
    h = _softplus(jnp.dot(h, w2_ref[...], preferred_element_type=jnp.float32)
                  + b2_ref[...])
    out_ref[...] = (jnp.dot(h, w3_ref[...], preferred_element_type=jnp.float32)
                    + b3_ref[...])


def _mlp_tail(hpre, W2, b2, W3, b3):
    BE = 3200
    grid = (N_EDGES // BE,)
    return pl.pallas_call(
        _mlp_body,
        grid=grid,
        in_specs=[
            pl.BlockSpec((BE, D), lambda i: (i, 0)),
            pl.BlockSpec((D, D), lambda i: (0, 0)),
            pl.BlockSpec((1, D), lambda i: (0, 0)),
            pl.BlockSpec((D, D), lambda i: (0, 0)),
            pl.BlockSpec((1, D), lambda i: (0, 0)),
        ],
        out_specs=pl.BlockSpec((BE, D), lambda i: (i, 0)),
        out_shape=jax.ShapeDtypeStruct((N_EDGES, D), jnp.float32),
    )(hpre, W2, b2.reshape(1, D), W3, b3.reshape(1, D))


def _scatter_body(h3_hbm, ridx_hbm, out_hbm, ix, bf, zbuf_v, accum_sh,
                  sli, slh, ss):
    cid = lax.axis_index("c")
    sid = lax.axis_index("s")
    wid = sid * NC + cid

    zeros16 = jnp.zeros((16,), jnp.float32)
    for i in range(8):
        for j in range(D // 16):
            zbuf_v[i, pl.ds(j * 16, 16)] = zeros16
    r0 = sid * N_PER_TILE

    def zchunk(k, c2):
        pltpu.sync_copy(zbuf_v, accum_sh.at[pl.ds(r0 + k * 8, 8)])
        return c2

    lax.fori_loop(0, N_PER_TILE // 8, zchunk, 0)
    plsc.subcore_barrier()

    e0 = wid * EPW

    def issue_load(ci, b):
        base = e0 + ci * C
        pltpu.async_copy(ridx_hbm.at[pl.ds(base, C)], ix[b], sli[b])
        pltpu.async_copy(h3_hbm.at[pl.ds(base, C)], bf[b], slh[b])

    def fire(b):
        pltpu.make_async_copy(ridx_hbm.at[pl.ds(0, C)], ix[b], sli[b]).wait()
        pltpu.make_async_copy(h3_hbm.at[pl.ds(0, C)], bf[b], slh[b]).wait()
        pltpu.async_copy(bf[b], accum_sh.at[ix[b]], ss[b], add=True)

    def drain(b):
        pltpu.make_async_copy(h3_hbm.at[pl.ds(0, C)], bf[b], ss[b]).wait()

    for b in range(NBUF):
        issue_load(b, b)

    def body(k, carry):
        g = k * NBUF
        for b in range(NBUF):
            fire(b)
        for b in range(NBUF):
            drain(b)

            @pl.when(g + NBUF + b < NCH)
            def _():
                issue_load(g + NBUF + b, b)

        return carry

    lax.fori_loop(0, (NCH - 1) // NBUF, body, 0)
    fire(0)
    drain(0)
    plsc.subcore_barrier()

    pltpu.sync_copy(accum_sh.at[pl.ds(r0, N_PER_TILE)],
                    out_hbm.at[cid, pl.ds(r0, N_PER_TILE)])


def _segment_sum(h3, ridx):
    mesh = plsc.VectorSubcoreMesh(core_axis_name="c", subcore_axis_name="s")
    f = pl.kernel(
        lambda h3r, rir, out, *rest: _scatter_body(
            h3r, rir, out,
            rest[0:NBUF], rest[NBUF:2 * NBUF],
            rest[2 * NBUF], rest[2 * NBUF + 1],
            rest[2 * NBUF + 2:2 * NBUF + 2 + NBUF],
            rest[2 * NBUF + 2 + NBUF:2 * NBUF + 2 + 2 * NBUF],
            rest[2 * NBUF + 2 + 2 * NBUF:2 * NBUF + 2 + 3 * NBUF]),
        out_type=jax.ShapeDtypeStruct((NC, NP, D), jnp.float32),
        mesh=mesh,
        scratch_types=(
            [pltpu.VMEM((C,), jnp.int32)] * NBUF
            + [pltpu.VMEM((C, D), jnp.float32)] * NBUF
            + [pltpu.VMEM((8, D), jnp.float32),
               pltpu.VMEM_SHARED((NP, D), jnp.float32)]
            + [pltpu.SemaphoreType.DMA] * (3 * NBUF)
        ),
    )
    return f(h3, ridx)


def _sum2_body(p_ref, o_ref):
    o_ref[...] = p_ref[0] + p_ref[1]


def _sum_partials(parts):
    BN = 632
    grid = (NP // BN,)
    return pl.pallas_call(
        _sum2_body,
        grid=grid,
        in_specs=[pl.BlockSpec((2, BN, D), lambda i: (0, i, 0))],
        out_specs=pl.BlockSpec((BN, D), lambda i: (i, 0)),
        out_shape=jax.ShapeDtypeStruct((NP, D), jnp.float32),
    )(parts)


def kernel(x, edge_idx, W1, b1, W2, b2, W3, b3):
    ridx = edge_idx[0].astype(jnp.int32)
    cidxp = edge_idx[1].astype(jnp.int32) + N_NODES

    xab_b = _preproj(x, W1, b1)
    xab_i = jax.lax.bitcast_convert_type(
        xab_b.reshape(2 * N_NODES, DW, 2), jnp.int32)
    hpre_i = _gather_pairs(xab_i, ridx, cidxp)
    hpre = jax.lax.bitcast_convert_type(
        hpre_i, jnp.bfloat16).reshape(N_EDGES, D)
    h3 = _mlp_tail(hpre, W2, b2, W3, b3)
    parts = _segment_sum(h3, ridx)
    return _sum_partials(parts)[:N_NODES]


# speedup vs baseline: 3.4058x; 3.4058x over previous
import jax
import jax.numpy as jnp
from jax import lax
from jax.experimental import pallas as pl
from jax.experimental.pallas import tpu as pltpu
from jax.experimental.pallas import tpu_sc as plsc

N_NODES = 10000
N_EDGES = 320000
D = 128

NC = 2
NS = 16
NW = NC * NS
EPW = N_EDGES // NW
C = 80
NCH = EPW // C
NP = 10112
N_PER_TILE = NP // NS


def _preproj_body(x_ref, w1_ref, b1_ref, out_ref):
    xa = jnp.dot(x_ref[...], w1_ref[:D, :], preferred_element_type=jnp.float32)
    xb = jnp.dot(x_ref[...], w1_ref[D:, :], preferred_element_type=jnp.float32)
    out_ref[0] = xa
    out_ref[1] = xb + b1_ref[...]


def _preproj(x, W1, b1):
    BN = 2000
    grid = (N_NODES // BN,)
    return pl.pallas_call(
        _preproj_body,
        grid=grid,
        in_specs=[
            pl.BlockSpec((BN, D), lambda i: (i, 0)),
            pl.BlockSpec((2 * D, D), lambda i: (0, 0)),
            pl.BlockSpec((1, D), lambda i: (0, 0)),
        ],
        out_specs=pl.BlockSpec((2, BN, D), lambda i: (0, i, 0)),
        out_shape=jax.ShapeDtypeStruct((2, N_NODES, D), jnp.float32),
    )(x, W1, b1.reshape(1, D))


NBUF = 4
NBUF2 = 5


def _gather_body(tab_hbm, ridx_hbm, cidx_hbm, out_hbm,
                 ia_v, ib_v, ba, bb, sga, sgb, ssa):
    wid = lax.axis_index("s") * NC + lax.axis_index("c")
    e0 = wid * EPW

    pltpu.sync_copy(ridx_hbm.at[pl.ds(e0, EPW)], ia_v)
    pltpu.sync_copy(cidx_hbm.at[pl.ds(e0, EPW)], ib_v)

    def issue_gather(ci, b):
        pltpu.async_copy(tab_hbm.at[ia_v.at[pl.ds(ci * C, C)]], ba[b], sga[b])
        pltpu.async_copy(tab_hbm.at[ib_v.at[pl.ds(ci * C, C)]], bb[b], sgb[b])

    def wait_gather(b):
        pltpu.make_async_copy(tab_hbm.at[pl.ds(0, C)], ba[b], sga[b]).wait()
        pltpu.make_async_copy(tab_hbm.at[pl.ds(0, C)], bb[b], sgb[b]).wait()

    def add_rows(b):
        A, B = ba[b], bb[b]

        def row_add(i, c2):
            for j in range(D // 16):
                sl = pl.ds(j * 16, 16)
                A[i, sl] = A[i, sl] + B[i, sl]
            return c2

        lax.fori_loop(0, C, row_add, 0, unroll=4)

    def issue_store(ci, b):
        base = e0 + ci * C
        pltpu.async_copy(ba[b], out_hbm.at[pl.ds(base, C)], ssa[b])

    def drain_store(b):
        pltpu.make_async_copy(ba[b], out_hbm.at[pl.ds(0, C)], ssa[b]).wait()

    for b in range(NBUF2):
        issue_store(b, b)

    def body(k, carry):
        g = k * NBUF2
        for b in range(NBUF2):
            drain_store(b)
            issue_gather(g + b, b)
        for b in range(NBUF2):
            wait_gather(b)
            add_rows(b)
            issue_store(g + b, b)
        return carry

    lax.fori_loop(0, NCH // NBUF2, body, 0)
    for b in range(NBUF2):
        drain_store(b)


def _gather_pairs(xab, ridx, cidxp):
    mesh = plsc.VectorSubcoreMesh(core_axis_name="c", subcore_axis_name="s")
    f = pl.kernel(
        lambda tab, ri, ci, out, iav, ibv, *rest: _gather_body(
            tab, ri, ci, out, iav, ibv,
            rest[0:NBUF2], rest[NBUF2:2 * NBUF2],
            rest[2 * NBUF2:3 * NBUF2], rest[3 * NBUF2:4 * NBUF2],
            rest[4 * NBUF2:5 * NBUF2]),
        out_type=jax.ShapeDtypeStruct((N_EDGES, D), jnp.float32),
        mesh=mesh,
        scratch_types=(
            [pltpu.VMEM((EPW,), jnp.int32)] * 2
            + [pltpu.VMEM((C, D), jnp.float32)] * (2 * NBUF2)
            + [pltpu.SemaphoreType.DMA] * (3 * NBUF2)
        ),
    )
    return f(xab, ridx, cidxp)


def _softplus(h):
    return jnp.maximum(h, 0.0) + jnp.log(1.0 + jnp.exp(-jnp.abs(h)))


def _mlp_body(h_ref, w2_ref, b2_ref, w3_ref, b3_ref, out_ref):
    h = _softplus(h_ref[...]).astype(jnp.bfloat16)
    h = _softplus(jnp.dot(h, w2_ref[...], preferred_element_type=jnp.float32)
                  + b2_ref[...]).astype(jnp.bfloat16)
    out_ref[...] = (jnp.dot(h, w3_ref[...], preferred_element_type=jnp.float32)
                    + b3_ref[...])


def _mlp_tail(hpre, W2, b2, W3, b3):
    BE = 3200
    grid = (N_EDGES // BE,)
    return pl.pallas_call(
        _mlp_body,
        grid=grid,
        in_specs=[
            pl.BlockSpec((BE, D), lambda i: (i, 0)),
            pl.BlockSpec((D, D), lambda i: (0, 0)),
            pl.BlockSpec((1, D), lambda i: (0, 0)),
            pl.BlockSpec((D, D), lambda i: (0, 0)),
            pl.BlockSpec((1, D), lambda i: (0, 0)),
        ],
        out_specs=pl.BlockSpec((BE, D), lambda i: (i, 0)),
        out_shape=jax.ShapeDtypeStruct((N_EDGES, D), jnp.float32),
    )(hpre, W2.astype(jnp.bfloat16), b2.reshape(1, D),
      W3.astype(jnp.bfloat16), b3.reshape(1, D))


def _scatter_body(h3_hbm, ridx_hbm, out_hbm, ix, bf, zbuf_v, accum_sh,
                  sli, slh, ss):
    cid = lax.axis_index("c")
    sid = lax.axis_index("s")
    wid = sid * NC + cid

    zeros16 = jnp.zeros((16,), jnp.float32)
    for i in range(8):
        for j in range(D // 16):
            zbuf_v[i, pl.ds(j * 16, 16)] = zeros16
    r0 = sid * N_PER_TILE

    def zchunk(k, c2):
        pltpu.sync_copy(zbuf_v, accum_sh.at[pl.ds(r0 + k * 8, 8)])
        return c2

    lax.fori_loop(0, N_PER_TILE // 8, zchunk, 0)
    plsc.subcore_barrier()

    e0 = wid * EPW

    def issue_load(ci, b):
        base = e0 + ci * C
        pltpu.async_copy(ridx_hbm.at[pl.ds(base, C)], ix[b], sli[b])
        pltpu.async_copy(h3_hbm.at[pl.ds(base, C)], bf[b], slh[b])

    def fire(b):
        pltpu.make_async_copy(ridx_hbm.at[pl.ds(0, C)], ix[b], sli[b]).wait()
        pltpu.make_async_copy(h3_hbm.at[pl.ds(0, C)], bf[b], slh[b]).wait()
        pltpu.async_copy(bf[b], accum_sh.at[ix[b]], ss[b], add=True)

    def drain(b):
        pltpu.make_async_copy(h3_hbm.at[pl.ds(0, C)], bf[b], ss[b]).wait()

    for b in range(NBUF):
        issue_load(b, b)

    def body(k, carry):
        g = k * NBUF
        for b in range(NBUF):
            fire(b)
        for b in range(NBUF):
            drain(b)

            @pl.when(g + NBUF + b < NCH)
            def _():
                issue_load(g + NBUF + b, b)

        return carry

    lax.fori_loop(0, (NCH - 1) // NBUF, body, 0)
    fire(0)
    drain(0)
    plsc.subcore_barrier()

    pltpu.sync_copy(accum_sh.at[pl.ds(r0, N_PER_TILE)],
                    out_hbm.at[cid, pl.ds(r0, N_PER_TILE)])


def _segment_sum(h3, ridx):
    mesh = plsc.VectorSubcoreMesh(core_axis_name="c", subcore_axis_name="s")
    f = pl.kernel(
        lambda h3r, rir, out, *rest: _scatter_body(
            h3r, rir, out,
            rest[0:NBUF], rest[NBUF:2 * NBUF],
            rest[2 * NBUF], rest[2 * NBUF + 1],
            rest[2 * NBUF + 2:2 * NBUF + 2 + NBUF],
            rest[2 * NBUF + 2 + NBUF:2 * NBUF + 2 + 2 * NBUF],
            rest[2 * NBUF + 2 + 2 * NBUF:2 * NBUF + 2 + 3 * NBUF]),
        out_type=jax.ShapeDtypeStruct((NC, NP, D), jnp.float32),
        mesh=mesh,
        scratch_types=(
            [pltpu.VMEM((C,), jnp.int32)] * NBUF
            + [pltpu.VMEM((C, D), jnp.float32)] * NBUF
            + [pltpu.VMEM((8, D), jnp.float32),
               pltpu.VMEM_SHARED((NP, D), jnp.float32)]
            + [pltpu.SemaphoreType.DMA] * (3 * NBUF)
        ),
    )
    return f(h3, ridx)


def _sum2_body(p_ref, o_ref):
    o_ref[...] = p_ref[0] + p_ref[1]


def _sum_partials(parts):
    BN = 632
    grid = (NP // BN,)
    return pl.pallas_call(
        _sum2_body,
        grid=grid,
        in_specs=[pl.BlockSpec((2, BN, D), lambda i: (0, i, 0))],
        out_specs=pl.BlockSpec((BN, D), lambda i: (i, 0)),
        out_shape=jax.ShapeDtypeStruct((NP, D), jnp.float32),
    )(parts)


def kernel(x, edge_idx, W1, b1, W2, b2, W3, b3):
    ridx = edge_idx[0].astype(jnp.int32)
    cidxp = edge_idx[1].astype(jnp.int32) + N_NODES

    xab = _preproj(x, W1, b1).reshape(2 * N_NODES, D)
    hpre = _gather_pairs(xab, ridx, cidxp)
    h3 = _mlp_tail(hpre, W2, b2, W3, b3)
    parts = _segment_sum(h3, ridx)
    return _sum_partials(parts)[:N_NODES]


# speedup vs baseline: 3.8220x; 1.1222x over previous
import jax
import jax.numpy as jnp
from jax import lax
from jax.experimental import pallas as pl
from jax.experimental.pallas import tpu as pltpu
from jax.experimental.pallas import tpu_sc as plsc

N_NODES = 10000
N_EDGES = 320000
D = 128

NC = 2
NS = 16
NW = NC * NS
NHALF = 2
EH = N_EDGES // NHALF
EPW = EH // NW
C = 40
NCH = EPW // C
NP = 10112
N_PER_TILE = NP // NS


def _preproj_body(x_ref, w1_ref, b1_ref, out_ref):
    xa = jnp.dot(x_ref[...], w1_ref[:D, :], preferred_element_type=jnp.float32)
    xb = jnp.dot(x_ref[...], w1_ref[D:, :], preferred_element_type=jnp.float32)
    out_ref[0] = xa
    out_ref[1] = xb + b1_ref[...]


def _preproj(x, W1, b1):
    BN = 2000
    grid = (N_NODES // BN,)
    return pl.pallas_call(
        _preproj_body,
        grid=grid,
        in_specs=[
            pl.BlockSpec((BN, D), lambda i: (i, 0)),
            pl.BlockSpec((2 * D, D), lambda i: (0, 0)),
            pl.BlockSpec((1, D), lambda i: (0, 0)),
        ],
        out_specs=pl.BlockSpec((2, BN, D), lambda i: (0, i, 0)),
        out_shape=jax.ShapeDtypeStruct((2, N_NODES, D), jnp.float32),
    )(x, W1, b1.reshape(1, D))


NBUF = 4
NBUF2 = 5


def _gather_body(tab_hbm, ridx_hbm, cidx_hbm, out_hbm,
                 ia_v, ib_v, ba, bb, sga, sgb, ssa):
    wid = lax.axis_index("s") * NC + lax.axis_index("c")
    e0 = wid * EPW

    pltpu.sync_copy(ridx_hbm.at[pl.ds(e0, EPW)], ia_v)
    pltpu.sync_copy(cidx_hbm.at[pl.ds(e0, EPW)], ib_v)

    def issue_gather(ci, b):
        pltpu.async_copy(tab_hbm.at[ia_v.at[pl.ds(ci * C, C)]], ba[b], sga[b])
        pltpu.async_copy(tab_hbm.at[ib_v.at[pl.ds(ci * C, C)]], bb[b], sgb[b])

    def wait_gather(b):
        pltpu.make_async_copy(tab_hbm.at[pl.ds(0, C)], ba[b], sga[b]).wait()
        pltpu.make_async_copy(tab_hbm.at[pl.ds(0, C)], bb[b], sgb[b]).wait()

    def add_rows(b):
        A, B = ba[b], bb[b]

        def row_add(i, c2):
            for j in range(D // 16):
                sl = pl.ds(j * 16, 16)
                A[i, sl] = A[i, sl] + B[i, sl]
            return c2

        lax.fori_loop(0, C, row_add, 0, unroll=4)

    def issue_store(ci, b):
        base = e0 + ci * C
        pltpu.async_copy(ba[b], out_hbm.at[pl.ds(base, C)], ssa[b])

    def drain_store(b):
        pltpu.make_async_copy(ba[b], out_hbm.at[pl.ds(0, C)], ssa[b]).wait()

    for b in range(NBUF2):
        issue_store(b, b)

    def body(k, carry):
        g = k * NBUF2
        for b in range(NBUF2):
            drain_store(b)
            issue_gather(g + b, b)
        for b in range(NBUF2):
            wait_gather(b)
            add_rows(b)
            issue_store(g + b, b)
        return carry

    lax.fori_loop(0, NCH // NBUF2, body, 0)
    for b in range(NBUF2):
        drain_store(b)


def _gather_pairs(xab, ridx, cidxp):
    mesh = plsc.VectorSubcoreMesh(core_axis_name="c", subcore_axis_name="s")
    f = pl.kernel(
        lambda tab, ri, ci, out, iav, ibv, *rest: _gather_body(
            tab, ri, ci, out, iav, ibv,
            rest[0:NBUF2], rest[NBUF2:2 * NBUF2],
            rest[2 * NBUF2:3 * NBUF2], rest[3 * NBUF2:4 * NBUF2],
            rest[4 * NBUF2:5 * NBUF2]),
        out_type=jax.ShapeDtypeStruct((EH, D), jnp.float32),
        mesh=mesh,
        scratch_types=(
            [pltpu.VMEM((EPW,), jnp.int32)] * 2
            + [pltpu.VMEM((C, D), jnp.float32)] * (2 * NBUF2)
            + [pltpu.SemaphoreType.DMA] * (3 * NBUF2)
        ),
    )
    return f(xab, ridx, cidxp)


def _softplus(h):
    return jnp.maximum(h, 0.0) + jnp.log(1.0 + jnp.exp(-jnp.abs(h)))


def _mlp_body(h_ref, w2_ref, b2_ref, w3_ref, b3_ref, out_ref):
    h = _softplus(h_ref[...]).astype(jnp.bfloat16)
    h = _softplus(jnp.dot(h, w2_ref[...], preferred_element_type=jnp.float32)
                  + b2_ref[...]).astype(jnp.bfloat16)
    out_ref[...] = (jnp.dot(h, w3_ref[...], preferred_element_type=jnp.float32)
                    + b3_ref[...])


def _mlp_tail(hpre, W2, b2, W3, b3):
    BE = 3200
    grid = (EH // BE,)
    return pl.pallas_call(
        _mlp_body,
        grid=grid,
        in_specs=[
            pl.BlockSpec((BE, D), lambda i: (i, 0)),
            pl.BlockSpec((D, D), lambda i: (0, 0)),
            pl.BlockSpec((1, D), lambda i: (0, 0)),
            pl.BlockSpec((D, D), lambda i: (0, 0)),
            pl.BlockSpec((1, D), lambda i: (0, 0)),
        ],
        out_specs=pl.BlockSpec((BE, D), lambda i: (i, 0)),
        out_shape=jax.ShapeDtypeStruct((EH, D), jnp.float32),
    )(hpre, W2.astype(jnp.bfloat16), b2.reshape(1, D),
      W3.astype(jnp.bfloat16), b3.reshape(1, D))


def _scatter_body(h3_hbm, ridx_hbm, out_hbm, ix, bf, zbuf_v, accum_sh,
                  sli, slh, ss):
    cid = lax.axis_index("c")
    sid = lax.axis_index("s")
    wid = sid * NC + cid

    zeros16 = jnp.zeros((16,), jnp.float32)
    for i in range(8):
        for j in range(D // 16):
            zbuf_v[i, pl.ds(j * 16, 16)] = zeros16
    r0 = sid * N_PER_TILE

    def zchunk(k, c2):
        pltpu.sync_copy(zbuf_v, accum_sh.at[pl.ds(r0 + k * 8, 8)])
        return c2

    lax.fori_loop(0, N_PER_TILE // 8, zchunk, 0)
    plsc.subcore_barrier()

    e0 = wid * EPW

    def issue_load(ci, b):
        base = e0 + ci * C
        pltpu.async_copy(ridx_hbm.at[pl.ds(base, C)], ix[b], sli[b])
        pltpu.async_copy(h3_hbm.at[pl.ds(base, C)], bf[b], slh[b])

    def fire(b):
        pltpu.make_async_copy(ridx_hbm.at[pl.ds(0, C)], ix[b], sli[b]).wait()
        pltpu.make_async_copy(h3_hbm.at[pl.ds(0, C)], bf[b], slh[b]).wait()
        pltpu.async_copy(bf[b], accum_sh.at[ix[b]], ss[b], add=True)

    def drain(b):
        pltpu.make_async_copy(h3_hbm.at[pl.ds(0, C)], bf[b], ss[b]).wait()

    for b in range(NBUF):
        issue_load(b, b)

    def body(k, carry):
        g = k * NBUF
        for b in range(NBUF):
            fire(b)
        for b in range(NBUF):
            drain(b)

            @pl.when(g + NBUF + b < NCH)
            def _():
                issue_load(g + NBUF + b, b)

        return carry

    lax.fori_loop(0, (NCH - 1) // NBUF, body, 0)
    fire(0)
    drain(0)
    plsc.subcore_barrier()

    pltpu.sync_copy(accum_sh.at[pl.ds(r0, N_PER_TILE)],
                    out_hbm.at[cid, pl.ds(r0, N_PER_TILE)])


def _segment_sum(h3, ridx):
    mesh = plsc.VectorSubcoreMesh(core_axis_name="c", subcore_axis_name="s")
    f = pl.kernel(
        lambda h3r, rir, out, *rest: _scatter_body(
            h3r, rir, out,
            rest[0:NBUF], rest[NBUF:2 * NBUF],
            rest[2 * NBUF], rest[2 * NBUF + 1],
            rest[2 * NBUF + 2:2 * NBUF + 2 + NBUF],
            rest[2 * NBUF + 2 + NBUF:2 * NBUF + 2 + 2 * NBUF],
            rest[2 * NBUF + 2 + 2 * NBUF:2 * NBUF + 2 + 3 * NBUF]),
        out_type=jax.ShapeDtypeStruct((NC, NP, D), jnp.float32),
        mesh=mesh,
        scratch_types=(
            [pltpu.VMEM((C,), jnp.int32)] * NBUF
            + [pltpu.VMEM((C, D), jnp.float32)] * NBUF
            + [pltpu.VMEM((8, D), jnp.float32),
               pltpu.VMEM_SHARED((NP, D), jnp.float32)]
            + [pltpu.SemaphoreType.DMA] * (3 * NBUF)
        ),
    )
    return f(h3, ridx)


def _sum2_body(pa_ref, pb_ref, o_ref):
    o_ref[...] = (pa_ref[0] + pa_ref[1]) + (pb_ref[0] + pb_ref[1])


def _sum_partials(pa, pb):
    BN = 632
    grid = (NP // BN,)
    return pl.pallas_call(
        _sum2_body,
        grid=grid,
        in_specs=[pl.BlockSpec((2, BN, D), lambda i: (0, i, 0)),
                  pl.BlockSpec((2, BN, D), lambda i: (0, i, 0))],
        out_specs=pl.BlockSpec((BN, D), lambda i: (i, 0)),
        out_shape=jax.ShapeDtypeStruct((NP, D), jnp.float32),
    )(pa, pb)


def kernel(x, edge_idx, W1, b1, W2, b2, W3, b3):
    ridx = edge_idx[0].astype(jnp.int32)
    cidxp = edge_idx[1].astype(jnp.int32) + N_NODES

    xab = _preproj(x, W1, b1).reshape(2 * N_NODES, D)
    parts = []
    for h in range(NHALF):
        rh = lax.dynamic_slice_in_dim(ridx, h * EH, EH)
        ch = lax.dynamic_slice_in_dim(cidxp, h * EH, EH)
        hpre = _gather_pairs(xab, rh, ch)
        h3 = _mlp_tail(hpre, W2, b2, W3, b3)
        parts.append(_segment_sum(h3, rh))
    return _sum_partials(parts[0], parts[1])[:N_NODES]


# speedup vs baseline: 4.0849x; 1.0688x over previous
import jax
import jax.numpy as jnp
from jax import lax
from jax.experimental import pallas as pl
from jax.experimental.pallas import tpu as pltpu
from jax.experimental.pallas import tpu_sc as plsc

N_NODES = 10000
N_EDGES = 320000
D = 128

NC = 2
NS = 16
NW = NC * NS
NHALF = 2
EH = N_EDGES // NHALF
EPW = EH // NW
C = 40
NCH = EPW // C
NP = 10112
N_PER_TILE = NP // NS


def _preproj_body(x_ref, w1_ref, b1_ref, out_ref):
    xa = jnp.dot(x_ref[...], w1_ref[:D, :], preferred_element_type=jnp.float32)
    xb = jnp.dot(x_ref[...], w1_ref[D:, :], preferred_element_type=jnp.float32)
    out_ref[0] = xa
    out_ref[1] = xb + b1_ref[...]


def _preproj(x, W1, b1):
    BN = 2000
    grid = (N_NODES // BN,)
    return pl.pallas_call(
        _preproj_body,
        grid=grid,
        in_specs=[
            pl.BlockSpec((BN, D), lambda i: (i, 0)),
            pl.BlockSpec((2 * D, D), lambda i: (0, 0)),
            pl.BlockSpec((1, D), lambda i: (0, 0)),
        ],
        out_specs=pl.BlockSpec((2, BN, D), lambda i: (0, i, 0)),
        out_shape=jax.ShapeDtypeStruct((2, N_NODES, D), jnp.float32),
    )(x, W1, b1.reshape(1, D))


NBUF = 4
NBUF2 = 5


DW = D // 2
PC = 125
TROWS = 2 * N_NODES // NS


def _gather_body(tab_hbm, ridx_hbm, cidx_hbm, out_hbm, ptab_hbm,
                 ia_v, ib_v, tf_v, tp_v, ba, bb, bo, ssa):
    cid = lax.axis_index("c")
    sid = lax.axis_index("s")
    wid = sid * NC + cid
    e0 = wid * EPW

    tr0 = sid * TROWS

    def pack_chunk(k, c2):
        r0 = tr0 + k * PC
        pltpu.sync_copy(tab_hbm.at[pl.ds(r0, PC)], tf_v)

        def prow(i, c3):
            for j in range(D // 32):
                a = tf_v[i, pl.ds(j * 32, 16)]
                b = tf_v[i, pl.ds(j * 32 + 16, 16)]
                w = plsc.bitcast(
                    plsc.pack(a, b, format=plsc.PackFormat.INTERLEAVED),
                    jnp.int32)
                tp_v[i, pl.ds(j * 16, 16)] = w
            return c3

        lax.fori_loop(0, PC, prow, 0, unroll=2)
        pltpu.sync_copy(tp_v, ptab_hbm.at[cid, pl.ds(r0, PC)])
        return c2

    lax.fori_loop(0, TROWS // PC, pack_chunk, 0)
    plsc.subcore_barrier()

    ptab = ptab_hbm.at[cid]

    pltpu.sync_copy(ridx_hbm.at[pl.ds(e0, EPW)], ia_v)
    pltpu.sync_copy(cidx_hbm.at[pl.ds(e0, EPW)], ib_v)

    def issue_gather(ci, b):
        pltpu.async_copy(ptab.at[ia_v.at[pl.ds(ci * C, C)]], ba[b], ssa[b])
        pltpu.async_copy(ptab.at[ib_v.at[pl.ds(ci * C, C)]], bb[b], ssa[b])

    def wait_gather(b):
        pltpu.make_async_copy(ptab.at[pl.ds(0, C)], ba[b], ssa[b]).wait()
        pltpu.make_async_copy(ptab.at[pl.ds(0, C)], bb[b], ssa[b]).wait()

    def add_unpack(b):
        A, B, O = ba[b], bb[b], bo[b]

        def row_add(i, c2):
            for j in range(DW // 16):
                sl = pl.ds(j * 16, 16)
                s = (plsc.bitcast(A[i, sl], jnp.bfloat16)
                     + plsc.bitcast(B[i, sl], jnp.bfloat16))
                lo, hi = plsc.unpack(s, format=plsc.PackFormat.INTERLEAVED)
                O[i, pl.ds(j * 32, 16)] = lo
                O[i, pl.ds(j * 32 + 16, 16)] = hi
            return c2

        lax.fori_loop(0, C, row_add, 0, unroll=2)

    def issue_store(ci, b):
        base = e0 + ci * C
        pltpu.async_copy(bo[b], out_hbm.at[pl.ds(base, C)], ssa[NBUF2 + b])

    def drain_store(b):
        pltpu.make_async_copy(bo[b], out_hbm.at[pl.ds(0, C)],
                              ssa[NBUF2 + b]).wait()

    for b in range(NBUF2):
        issue_store(b, b)

    def body(k, carry):
        g = k * NBUF2
        for b in range(NBUF2):
            issue_gather(g + b, b)
        for b in range(NBUF2):
            wait_gather(b)
            drain_store(b)
            add_unpack(b)
            issue_store(g + b, b)
        return carry

    lax.fori_loop(0, NCH // NBUF2, body, 0)
    for b in range(NBUF2):
        drain_store(b)


def _gather_pairs(xab, ridx, cidxp):
    mesh = plsc.VectorSubcoreMesh(core_axis_name="c", subcore_axis_name="s")
    f = pl.kernel(
        lambda tab, ri, ci, out, ptab, iav, ibv, tfv, tpv, *rest: _gather_body(
            tab, ri, ci, out, ptab, iav, ibv, tfv, tpv,
            rest[0:NBUF2], rest[NBUF2:2 * NBUF2],
            rest[2 * NBUF2:3 * NBUF2], rest[3 * NBUF2:]),
        out_type=(jax.ShapeDtypeStruct((EH, D), jnp.float32),
                  jax.ShapeDtypeStruct((NC, 2 * N_NODES, DW), jnp.int32)),
        mesh=mesh,
        compiler_params=pltpu.CompilerParams(use_tc_tiling_on_sc=False,
                                             needs_layout_passes=False),
        scratch_types=(
            [pltpu.VMEM((EPW,), jnp.int32)] * 2
            + [pltpu.VMEM((PC, D), jnp.float32),
               pltpu.VMEM((PC, DW), jnp.int32)]
            + [pltpu.VMEM((C, DW), jnp.int32)] * (2 * NBUF2)
            + [pltpu.VMEM((C, D), jnp.float32)] * NBUF2
            + [pltpu.SemaphoreType.DMA] * (2 * NBUF2)
        ),
    )
    hpre, _ = f(xab, ridx, cidxp)
    return hpre


def _softplus(h):
    return jnp.maximum(h, 0.0) + jnp.log(1.0 + jnp.exp(-jnp.abs(h)))


def _mlp_body(h_ref, w2_ref, b2_ref, w3_ref, b3_ref, out_ref):
    h = _softplus(h_ref[...]).astype(jnp.bfloat16)
    h = _softplus(jnp.dot(h, w2_ref[...], preferred_element_type=jnp.float32)
                  + b2_ref[...]).astype(jnp.bfloat16)
    out_ref[...] = (jnp.dot(h, w3_ref[...], preferred_element_type=jnp.float32)
                    + b3_ref[...])


def _mlp_tail(hpre, W2, b2, W3, b3):
    BE = 3200
    grid = (EH // BE,)
    return pl.pallas_call(
        _mlp_body,
        grid=grid,
        in_specs=[
            pl.BlockSpec((BE, D), lambda i: (i, 0)),
            pl.BlockSpec((D, D), lambda i: (0, 0)),
            pl.BlockSpec((1, D), lambda i: (0, 0)),
            pl.BlockSpec((D, D), lambda i: (0, 0)),
            pl.BlockSpec((1, D), lambda i: (0, 0)),
        ],
        out_specs=pl.BlockSpec((BE, D), lambda i: (i, 0)),
        out_shape=jax.ShapeDtypeStruct((EH, D), jnp.float32),
    )(hpre, W2.astype(jnp.bfloat16), b2.reshape(1, D),
      W3.astype(jnp.bfloat16), b3.reshape(1, D))


def _scatter_body(h3_hbm, ridx_hbm, out_hbm, ix, bf, zbuf_v, accum_sh,
                  sli, slh, ss):
    cid = lax.axis_index("c")
    sid = lax.axis_index("s")
    wid = sid * NC + cid

    zeros16 = jnp.zeros((16,), jnp.float32)
    for i in range(8):
        for j in range(D // 16):
            zbuf_v[i, pl.ds(j * 16, 16)] = zeros16
    r0 = sid * N_PER_TILE

    def zchunk(k, c2):
        pltpu.sync_copy(zbuf_v, accum_sh.at[pl.ds(r0 + k * 8, 8)])
        return c2

    lax.fori_loop(0, N_PER_TILE // 8, zchunk, 0)
    plsc.subcore_barrier()

    e0 = wid * EPW

    def issue_load(ci, b):
        base = e0 + ci * C
        pltpu.async_copy(ridx_hbm.at[pl.ds(base, C)], ix[b], sli[b])
        pltpu.async_copy(h3_hbm.at[pl.ds(base, C)], bf[b], slh[b])

    def fire(b):
        pltpu.make_async_copy(ridx_hbm.at[pl.ds(0, C)], ix[b], sli[b]).wait()
        pltpu.make_async_copy(h3_hbm.at[pl.ds(0, C)], bf[b], slh[b]).wait()
        pltpu.async_copy(bf[b], accum_sh.at[ix[b]], ss[b], add=True)

    def drain(b):
        pltpu.make_async_copy(h3_hbm.at[pl.ds(0, C)], bf[b], ss[b]).wait()

    for b in range(NBUF):
        issue_load(b, b)

    def body(k, carry):
        g = k * NBUF
        for b in range(NBUF):
            fire(b)
        for b in range(NBUF):
            drain(b)

            @pl.when(g + NBUF + b < NCH)
            def _():
                issue_load(g + NBUF + b, b)

        return carry

    lax.fori_loop(0, (NCH - 1) // NBUF, body, 0)
    fire(0)
    drain(0)
    plsc.subcore_barrier()

    pltpu.sync_copy(accum_sh.at[pl.ds(r0, N_PER_TILE)],
                    out_hbm.at[cid, pl.ds(r0, N_PER_TILE)])


def _segment_sum(h3, ridx):
    mesh = plsc.VectorSubcoreMesh(core_axis_name="c", subcore_axis_name="s")
    f = pl.kernel(
        lambda h3r, rir, out, *rest: _scatter_body(
            h3r, rir, out,
            rest[0:NBUF], rest[NBUF:2 * NBUF],
            rest[2 * NBUF], rest[2 * NBUF + 1],
            rest[2 * NBUF + 2:2 * NBUF + 2 + NBUF],
            rest[2 * NBUF + 2 + NBUF:2 * NBUF + 2 + 2 * NBUF],
            rest[2 * NBUF + 2 + 2 * NBUF:2 * NBUF + 2 + 3 * NBUF]),
        out_type=jax.ShapeDtypeStruct((NC, NP, D), jnp.float32),
        mesh=mesh,
        scratch_types=(
            [pltpu.VMEM((C,), jnp.int32)] * NBUF
            + [pltpu.VMEM((C, D), jnp.float32)] * NBUF
            + [pltpu.VMEM((8, D), jnp.float32),
               pltpu.VMEM_SHARED((NP, D), jnp.float32)]
            + [pltpu.SemaphoreType.DMA] * (3 * NBUF)
        ),
    )
    return f(h3, ridx)


def _sum2_body(pa_ref, pb_ref, o_ref):
    o_ref[...] = (pa_ref[0] + pa_ref[1]) + (pb_ref[0] + pb_ref[1])


def _sum_partials(pa, pb):
    BN = 632
    grid = (NP // BN,)
    return pl.pallas_call(
        _sum2_body,
        grid=grid,
        in_specs=[pl.BlockSpec((2, BN, D), lambda i: (0, i, 0)),
                  pl.BlockSpec((2, BN, D), lambda i: (0, i, 0))],
        out_specs=pl.BlockSpec((BN, D), lambda i: (i, 0)),
        out_shape=jax.ShapeDtypeStruct((NP, D), jnp.float32),
    )(pa, pb)


def kernel(x, edge_idx, W1, b1, W2, b2, W3, b3):
    ridx = edge_idx[0].astype(jnp.int32)
    cidxp = edge_idx[1].astype(jnp.int32) + N_NODES

    xab = _preproj(x, W1, b1).reshape(2 * N_NODES, D)
    parts = []
    for h in range(NHALF):
        rh = lax.dynamic_slice_in_dim(ridx, h * EH, EH)
        ch = lax.dynamic_slice_in_dim(cidxp, h * EH, EH)
        hpre = _gather_pairs(xab, rh, ch)
        h3 = _mlp_tail(hpre, W2, b2, W3, b3)
        parts.append(_segment_sum(h3, rh))
    return _sum_partials(parts[0], parts[1])[:N_NODES]


# speedup vs baseline: 4.3033x; 1.0535x over previous
import jax
import jax.numpy as jnp
from jax import lax
from jax.experimental import pallas as pl
from jax.experimental.pallas import tpu as pltpu
from jax.experimental.pallas import tpu_sc as plsc

N_NODES = 10000
N_EDGES = 320000
D = 128

NC = 2
NS = 16
NW = NC * NS
NHALF = 2
EH = N_EDGES // NHALF
EPW = EH // NW
C = 40
NCH = EPW // C
NP = 10112
N_PER_TILE = NP // NS


def _preproj_body(x_ref, w1_ref, b1_ref, out_ref):
    xa = jnp.dot(x_ref[...], w1_ref[:D, :], preferred_element_type=jnp.float32)
    xb = jnp.dot(x_ref[...], w1_ref[D:, :], preferred_element_type=jnp.float32)
    out_ref[0] = xa
    out_ref[1] = xb + b1_ref[...]


def _preproj(x, W1, b1):
    BN = 2000
    grid = (N_NODES // BN,)
    return pl.pallas_call(
        _preproj_body,
        grid=grid,
        in_specs=[
            pl.BlockSpec((BN, D), lambda i: (i, 0)),
            pl.BlockSpec((2 * D, D), lambda i: (0, 0)),
            pl.BlockSpec((1, D), lambda i: (0, 0)),
        ],
        out_specs=pl.BlockSpec((2, BN, D), lambda i: (0, i, 0)),
        out_shape=jax.ShapeDtypeStruct((2, N_NODES, D), jnp.float32),
    )(x, W1, b1.reshape(1, D))


NBUF = 4
NBUF2 = 5


DW = D // 2
PC = 125
TROWS = 2 * N_NODES // NS


def _gather_body(do_pack, tab_hbm, ridx_hbm, cidx_hbm, out_hbm, ptab_hbm,
                 ia_v, ib_v, tf_v, tp_v, ba, bb, bo, ssa):
    cid = lax.axis_index("c")
    sid = lax.axis_index("s")
    wid = sid * NC + cid
    e0 = wid * EPW

    if do_pack:
        tr0 = sid * TROWS

        def pack_chunk(k, c2):
            r0 = tr0 + k * PC
            pltpu.sync_copy(tab_hbm.at[pl.ds(r0, PC)], tf_v)

            def prow(i, c3):
                for j in range(D // 32):
                    a = tf_v[i, pl.ds(j * 32, 16)]
                    b = tf_v[i, pl.ds(j * 32 + 16, 16)]
                    w = plsc.bitcast(
                        plsc.pack(a, b, format=plsc.PackFormat.INTERLEAVED),
                        jnp.int32)
                    tp_v[i, pl.ds(j * 16, 16)] = w
                return c3

            lax.fori_loop(0, PC, prow, 0, unroll=2)
            pltpu.sync_copy(tp_v, ptab_hbm.at[cid, pl.ds(r0, PC)])
            return c2

        lax.fori_loop(0, TROWS // PC, pack_chunk, 0)
        plsc.subcore_barrier()

    ptab = ptab_hbm.at[cid]

    pltpu.sync_copy(ridx_hbm.at[pl.ds(e0, EPW)], ia_v)
    pltpu.sync_copy(cidx_hbm.at[pl.ds(e0, EPW)], ib_v)

    def issue_gather(ci, b):
        pltpu.async_copy(ptab.at[ia_v.at[pl.ds(ci * C, C)]], ba[b], ssa[b])
        pltpu.async_copy(ptab.at[ib_v.at[pl.ds(ci * C, C)]], bb[b], ssa[b])

    def wait_gather(b):
        pltpu.make_async_copy(ptab.at[pl.ds(0, C)], ba[b], ssa[b]).wait()
        pltpu.make_async_copy(ptab.at[pl.ds(0, C)], bb[b], ssa[b]).wait()

    def add_unpack(b):
        A, B, O = ba[b], bb[b], bo[b]

        def row_add(i, c2):
            for j in range(DW // 16):
                sl = pl.ds(j * 16, 16)
                s = (plsc.bitcast(A[i, sl], jnp.bfloat16)
                     + plsc.bitcast(B[i, sl], jnp.bfloat16))
                lo, hi = plsc.unpack(s, format=plsc.PackFormat.INTERLEAVED)
                O[i, pl.ds(j * 32, 16)] = lo
                O[i, pl.ds(j * 32 + 16, 16)] = hi
            return c2

        lax.fori_loop(0, C, row_add, 0, unroll=4)

    def issue_store(ci, b):
        base = e0 + ci * C
        pltpu.async_copy(bo[b], out_hbm.at[pl.ds(base, C)], ssa[NBUF2 + b])

    def drain_store(b):
        pltpu.make_async_copy(bo[b], out_hbm.at[pl.ds(0, C)],
                              ssa[NBUF2 + b]).wait()

    for b in range(NBUF2):
        issue_store(b, b)

    def body(k, carry):
        g = k * NBUF2
        for b in range(NBUF2):
            issue_gather(g + b, b)
        for b in range(NBUF2):
            wait_gather(b)
            drain_store(b)
            add_unpack(b)
            issue_store(g + b, b)
        return carry

    lax.fori_loop(0, NCH // NBUF2, body, 0)
    for b in range(NBUF2):
        drain_store(b)


_GATHER_SCRATCH = (
    [pltpu.VMEM((EPW,), jnp.int32)] * 2
    + [pltpu.VMEM((PC, D), jnp.float32),
       pltpu.VMEM((PC, DW), jnp.int32)]
    + [pltpu.VMEM((C, DW), jnp.int32)] * (2 * NBUF2)
    + [pltpu.VMEM((C, D), jnp.float32)] * NBUF2
    + [pltpu.SemaphoreType.DMA] * (2 * NBUF2)
)
_SC_PARAMS = pltpu.CompilerParams(use_tc_tiling_on_sc=False,
                                  needs_layout_passes=False)


def _gather_pairs_pack(xab, ridx, cidxp):
    mesh = plsc.VectorSubcoreMesh(core_axis_name="c", subcore_axis_name="s")
    f = pl.kernel(
        lambda tab, ri, ci, out, ptab, iav, ibv, tfv, tpv, *rest: _gather_body(
            True, tab, ri, ci, out, ptab, iav, ibv, tfv, tpv,
            rest[0:NBUF2], rest[NBUF2:2 * NBUF2],
            rest[2 * NBUF2:3 * NBUF2], rest[3 * NBUF2:]),
        out_type=(jax.ShapeDtypeStruct((EH, D), jnp.float32),
                  jax.ShapeDtypeStruct((NC, 2 * N_NODES, DW), jnp.int32)),
        mesh=mesh,
        compiler_params=_SC_PARAMS,
        scratch_types=_GATHER_SCRATCH,
    )
    return f(xab, ridx, cidxp)


def _gather_pairs_reuse(ptab, ridx, cidxp):
    mesh = plsc.VectorSubcoreMesh(core_axis_name="c", subcore_axis_name="s")
    f = pl.kernel(
        lambda pt, ri, ci, out, iav, ibv, tfv, tpv, *rest: _gather_body(
            False, None, ri, ci, out, pt, iav, ibv, tfv, tpv,
            rest[0:NBUF2], rest[NBUF2:2 * NBUF2],
            rest[2 * NBUF2:3 * NBUF2], rest[3 * NBUF2:]),
        out_type=jax.ShapeDtypeStruct((EH, D), jnp.float32),
        mesh=mesh,
        compiler_params=_SC_PARAMS,
        scratch_types=_GATHER_SCRATCH,
    )
    return f(ptab, ridx, cidxp)


def _softplus(h):
    return jnp.maximum(h, 0.0) + jnp.log(1.0 + jnp.exp(-jnp.abs(h)))


def _mlp_body(h_ref, w2_ref, b2_ref, w3_ref, b3_ref, out_ref):
    h = _softplus(h_ref[...]).astype(jnp.bfloat16)
    h = _softplus(jnp.dot(h, w2_ref[...], preferred_element_type=jnp.float32)
                  + b2_ref[...]).astype(jnp.bfloat16)
    out_ref[...] = (jnp.dot(h, w3_ref[...], preferred_element_type=jnp.float32)
                    + b3_ref[...])


def _mlp_tail(hpre, W2, b2, W3, b3):
    BE = 3200
    grid = (EH // BE,)
    return pl.pallas_call(
        _mlp_body,
        grid=grid,
        in_specs=[
            pl.BlockSpec((BE, D), lambda i: (i, 0)),
            pl.BlockSpec((D, D), lambda i: (0, 0)),
            pl.BlockSpec((1, D), lambda i: (0, 0)),
            pl.BlockSpec((D, D), lambda i: (0, 0)),
            pl.BlockSpec((1, D), lambda i: (0, 0)),
        ],
        out_specs=pl.BlockSpec((BE, D), lambda i: (i, 0)),
        out_shape=jax.ShapeDtypeStruct((EH, D), jnp.float32),
    )(hpre, W2.astype(jnp.bfloat16), b2.reshape(1, D),
      W3.astype(jnp.bfloat16), b3.reshape(1, D))


def _scatter_body(h3_hbm, ridx_hbm, out_hbm, ix, bf, zbuf_v, accum_sh,
                  sli, slh, ss):
    cid = lax.axis_index("c")
    sid = lax.axis_index("s")
    wid = sid * NC + cid

    zeros16 = jnp.zeros((16,), jnp.float32)
    for i in range(8):
        for j in range(D // 16):
            zbuf_v[i, pl.ds(j * 16, 16)] = zeros16
    r0 = sid * N_PER_TILE

    def zchunk(k, c2):
        pltpu.sync_copy(zbuf_v, accum_sh.at[pl.ds(r0 + k * 8, 8)])
        return c2

    lax.fori_loop(0, N_PER_TILE // 8, zchunk, 0)
    plsc.subcore_barrier()

    e0 = wid * EPW

    def issue_load(ci, b):
        base = e0 + ci * C
        pltpu.async_copy(ridx_hbm.at[pl.ds(base, C)], ix[b], sli[b])
        pltpu.async_copy(h3_hbm.at[pl.ds(base, C)], bf[b], slh[b])

    def fire(b):
        pltpu.make_async_copy(ridx_hbm.at[pl.ds(0, C)], ix[b], sli[b]).wait()
        pltpu.make_async_copy(h3_hbm.at[pl.ds(0, C)], bf[b], slh[b]).wait()
        pltpu.async_copy(bf[b], accum_sh.at[ix[b]], ss[b], add=True)

    def drain(b):
        pltpu.make_async_copy(h3_hbm.at[pl.ds(0, C)], bf[b], ss[b]).wait()

    for b in range(NBUF):
        issue_load(b, b)

    def body(k, carry):
        g = k * NBUF
        for b in range(NBUF):
            fire(b)
        for b in range(NBUF):
            drain(b)

            @pl.when(g + NBUF + b < NCH)
            def _():
                issue_load(g + NBUF + b, b)

        return carry

    lax.fori_loop(0, (NCH - 1) // NBUF, body, 0)
    fire(0)
    drain(0)
    plsc.subcore_barrier()

    pltpu.sync_copy(accum_sh.at[pl.ds(r0, N_PER_TILE)],
                    out_hbm.at[cid, pl.ds(r0, N_PER_TILE)])


def _segment_sum(h3, ridx):
    mesh = plsc.VectorSubcoreMesh(core_axis_name="c", subcore_axis_name="s")
    f = pl.kernel(
        lambda h3r, rir, out, *rest: _scatter_body(
            h3r, rir, out,
            rest[0:NBUF], rest[NBUF:2 * NBUF],
            rest[2 * NBUF], rest[2 * NBUF + 1],
            rest[2 * NBUF + 2:2 * NBUF + 2 + NBUF],
            rest[2 * NBUF + 2 + NBUF:2 * NBUF + 2 + 2 * NBUF],
            rest[2 * NBUF + 2 + 2 * NBUF:2 * NBUF + 2 + 3 * NBUF]),
        out_type=jax.ShapeDtypeStruct((NC, NP, D), jnp.float32),
        mesh=mesh,
        scratch_types=(
            [pltpu.VMEM((C,), jnp.int32)] * NBUF
            + [pltpu.VMEM((C, D), jnp.float32)] * NBUF
            + [pltpu.VMEM((8, D), jnp.float32),
               pltpu.VMEM_SHARED((NP, D), jnp.float32)]
            + [pltpu.SemaphoreType.DMA] * (3 * NBUF)
        ),
    )
    return f(h3, ridx)


def _sum2_body(pa_ref, pb_ref, o_ref):
    o_ref[...] = (pa_ref[0] + pa_ref[1]) + (pb_ref[0] + pb_ref[1])


def _sum_partials(pa, pb):
    BN = 632
    grid = (NP // BN,)
    return pl.pallas_call(
        _sum2_body,
        grid=grid,
        in_specs=[pl.BlockSpec((2, BN, D), lambda i: (0, i, 0)),
                  pl.BlockSpec((2, BN, D), lambda i: (0, i, 0))],
        out_specs=pl.BlockSpec((BN, D), lambda i: (i, 0)),
        out_shape=jax.ShapeDtypeStruct((NP, D), jnp.float32),
    )(pa, pb)


def kernel(x, edge_idx, W1, b1, W2, b2, W3, b3):
    ridx = edge_idx[0].astype(jnp.int32)
    cidxp = edge_idx[1].astype(jnp.int32) + N_NODES

    xab = _preproj(x, W1, b1).reshape(2 * N_NODES, D)
    r0 = lax.dynamic_slice_in_dim(ridx, 0, EH)
    c0 = lax.dynamic_slice_in_dim(cidxp, 0, EH)
    r1 = lax.dynamic_slice_in_dim(ridx, EH, EH)
    c1 = lax.dynamic_slice_in_dim(cidxp, EH, EH)

    hpre0, ptab = _gather_pairs_pack(xab, r0, c0)
    h3_0 = _mlp_tail(hpre0, W2, b2, W3, b3)
    hpre1 = _gather_pairs_reuse(ptab, r1, c1)
    p0 = _segment_sum(h3_0, r0)
    h3_1 = _mlp_tail(hpre1, W2, b2, W3, b3)
    p1 = _segment_sum(h3_1, r1)
    return _sum_partials(p0, p1)[:N_NODES]


# speedup vs baseline: 4.3167x; 1.0031x over previous
import jax
import jax.numpy as jnp
from jax import lax
from jax.experimental import pallas as pl
from jax.experimental.pallas import tpu as pltpu
from jax.experimental.pallas import tpu_sc as plsc

N_NODES = 10000
N_EDGES = 320000
D = 128

NC = 2
NS = 16
NW = NC * NS
NHALF = 2
EH = N_EDGES // NHALF
EPW = EH // NW
C = 40
NCH = EPW // C
NP = 10112
N_PER_TILE = NP // NS


def _preproj_body(x_ref, w1_ref, b1_ref, out_ref):
    xa = jnp.dot(x_ref[...], w1_ref[:D, :], preferred_element_type=jnp.float32)
    xb = jnp.dot(x_ref[...], w1_ref[D:, :], preferred_element_type=jnp.float32)
    out_ref[0] = xa
    out_ref[1] = xb + b1_ref[...]


def _preproj(x, W1, b1):
    BN = 2000
    grid = (N_NODES // BN,)
    return pl.pallas_call(
        _preproj_body,
        grid=grid,
        in_specs=[
            pl.BlockSpec((BN, D), lambda i: (i, 0)),
            pl.BlockSpec((2 * D, D), lambda i: (0, 0)),
            pl.BlockSpec((1, D), lambda i: (0, 0)),
        ],
        out_specs=pl.BlockSpec((2, BN, D), lambda i: (0, i, 0)),
        out_shape=jax.ShapeDtypeStruct((2, N_NODES, D), jnp.float32),
    )(x, W1, b1.reshape(1, D))


NBUF = 4
NBUF2 = 5


DW = D // 2
PC = 125
TROWS = 2 * N_NODES // NS


def _gather_body(do_pack, tab_hbm, ridx_hbm, cidx_hbm, out_hbm, ptab_hbm,
                 ia_v, ib_v, tf_v, tp_v, ba, bb, bo, ssa):
    cid = lax.axis_index("c")
    sid = lax.axis_index("s")
    wid = sid * NC + cid
    e0 = wid * EPW

    if do_pack:
        tr0 = sid * TROWS

        def pack_chunk(k, c2):
            r0 = tr0 + k * PC
            pltpu.sync_copy(tab_hbm.at[pl.ds(r0, PC)], tf_v)

            def prow(i, c3):
                for j in range(D // 32):
                    a = tf_v[i, pl.ds(j * 32, 16)]
                    b = tf_v[i, pl.ds(j * 32 + 16, 16)]
                    w = plsc.bitcast(
                        plsc.pack(a, b, format=plsc.PackFormat.INTERLEAVED),
                        jnp.int32)
                    tp_v[i, pl.ds(j * 16, 16)] = w
                return c3

            lax.fori_loop(0, PC, prow, 0, unroll=2)
            pltpu.sync_copy(tp_v, ptab_hbm.at[cid, pl.ds(r0, PC)])
            return c2

        lax.fori_loop(0, TROWS // PC, pack_chunk, 0)
        plsc.subcore_barrier()

    ptab = ptab_hbm.at[cid]

    pltpu.sync_copy(ridx_hbm.at[pl.ds(e0, EPW)], ia_v)
    pltpu.sync_copy(cidx_hbm.at[pl.ds(e0, EPW)], ib_v)

    def issue_gather(ci, b):
        pltpu.async_copy(ptab.at[ia_v.at[pl.ds(ci * C, C)]], ba[b], ssa[b])
        pltpu.async_copy(ptab.at[ib_v.at[pl.ds(ci * C, C)]], bb[b], ssa[b])

    def wait_gather(b):
        pltpu.make_async_copy(ptab.at[pl.ds(0, C)], ba[b], ssa[b]).wait()
        pltpu.make_async_copy(ptab.at[pl.ds(0, C)], bb[b], ssa[b]).wait()

    def add_unpack(b):
        A, B, O = ba[b], bb[b], bo[b]

        def row_add(i, c2):
            for j in range(DW // 16):
                sl = pl.ds(j * 16, 16)
                s = (plsc.bitcast(A[i, sl], jnp.bfloat16)
                     + plsc.bitcast(B[i, sl], jnp.bfloat16))
                lo, hi = plsc.unpack(s, format=plsc.PackFormat.INTERLEAVED)
                O[i, pl.ds(j * 32, 16)] = lo
                O[i, pl.ds(j * 32 + 16, 16)] = hi
            return c2

        lax.fori_loop(0, C, row_add, 0, unroll=4)

    def issue_store(ci, b):
        base = e0 + ci * C
        pltpu.async_copy(bo[b], out_hbm.at[pl.ds(base, C)], ssa[NBUF2 + b])

    def drain_store(b):
        pltpu.make_async_copy(bo[b], out_hbm.at[pl.ds(0, C)],
                              ssa[NBUF2 + b]).wait()

    for b in range(NBUF2):
        issue_store(b, b)

    def body(k, carry):
        g = k * NBUF2
        for b in range(NBUF2):
            issue_gather(g + b, b)
        for b in range(NBUF2):
            wait_gather(b)
            drain_store(b)
            add_unpack(b)
            issue_store(g + b, b)
        return carry

    lax.fori_loop(0, NCH // NBUF2, body, 0)
    for b in range(NBUF2):
        drain_store(b)


_GATHER_SCRATCH = (
    [pltpu.VMEM((EPW,), jnp.int32)] * 2
    + [pltpu.VMEM((PC, D), jnp.float32),
       pltpu.VMEM((PC, DW), jnp.int32)]
    + [pltpu.VMEM((C, DW), jnp.int32)] * (2 * NBUF2)
    + [pltpu.VMEM((C, D), jnp.float32)] * NBUF2
    + [pltpu.SemaphoreType.DMA] * (2 * NBUF2)
)
_SC_PARAMS = pltpu.CompilerParams(use_tc_tiling_on_sc=False,
                                  needs_layout_passes=False)


def _gather_pairs_pack(xab, ridx, cidxp):
    mesh = plsc.VectorSubcoreMesh(core_axis_name="c", subcore_axis_name="s")
    f = pl.kernel(
        lambda tab, ri, ci, out, ptab, iav, ibv, tfv, tpv, *rest: _gather_body(
            True, tab, ri, ci, out, ptab, iav, ibv, tfv, tpv,
            rest[0:NBUF2], rest[NBUF2:2 * NBUF2],
            rest[2 * NBUF2:3 * NBUF2], rest[3 * NBUF2:]),
        out_type=(jax.ShapeDtypeStruct((EH, D), jnp.float32),
                  jax.ShapeDtypeStruct((NC, 2 * N_NODES, DW), jnp.int32)),
        mesh=mesh,
        compiler_params=_SC_PARAMS,
        scratch_types=_GATHER_SCRATCH,
    )
    return f(xab, ridx, cidxp)


def _gather_pairs_reuse(ptab, ridx, cidxp):
    mesh = plsc.VectorSubcoreMesh(core_axis_name="c", subcore_axis_name="s")
    f = pl.kernel(
        lambda pt, ri, ci, out, iav, ibv, tfv, tpv, *rest: _gather_body(
            False, None, ri, ci, out, pt, iav, ibv, tfv, tpv,
            rest[0:NBUF2], rest[NBUF2:2 * NBUF2],
            rest[2 * NBUF2:3 * NBUF2], rest[3 * NBUF2:]),
        out_type=jax.ShapeDtypeStruct((EH, D), jnp.float32),
        mesh=mesh,
        compiler_params=_SC_PARAMS,
        scratch_types=_GATHER_SCRATCH,
    )
    return f(ptab, ridx, cidxp)


def _softplus(h):
    return jnp.maximum(h, 0.0) + jnp.log(1.0 + jnp.exp(-jnp.abs(h)))


def _mlp_body(h_ref, w2_ref, b2_ref, w3_ref, b3_ref, out_ref):
    h = _softplus(h_ref[...]).astype(jnp.bfloat16)
    h = _softplus(jnp.dot(h, w2_ref[...], preferred_element_type=jnp.float32)
                  + b2_ref[...]).astype(jnp.bfloat16)
    out_ref[...] = (jnp.dot(h, w3_ref[...], preferred_element_type=jnp.float32)
                    + b3_ref[...])


def _mlp_tail(hpre, W2, b2, W3, b3):
    BE = 3200
    grid = (EH // BE,)
    return pl.pallas_call(
        _mlp_body,
        grid=grid,
        in_specs=[
            pl.BlockSpec((BE, D), lambda i: (i, 0)),
            pl.BlockSpec((D, D), lambda i: (0, 0)),
            pl.BlockSpec((1, D), lambda i: (0, 0)),
            pl.BlockSpec((D, D), lambda i: (0, 0)),
            pl.BlockSpec((1, D), lambda i: (0, 0)),
        ],
        out_specs=pl.BlockSpec((BE, D), lambda i: (i, 0)),
        out_shape=jax.ShapeDtypeStruct((EH, D), jnp.float32),
    )(hpre, W2.astype(jnp.bfloat16), b2.reshape(1, D),
      W3.astype(jnp.bfloat16), b3.reshape(1, D))


ZR = 64


def _scatter_body(h3_hbm, ridx3_hbm, out_hbm, ix2_v, bf, zbuf_v, accum_sh,
                  slh, ss):
    cid = lax.axis_index("c")
    sid = lax.axis_index("s")
    wid = sid * NC + cid

    zeros16 = jnp.zeros((16,), jnp.float32)
    for i in range(ZR):
        for j in range(D // 16):
            zbuf_v[i, pl.ds(j * 16, 16)] = zeros16
    r0 = sid * N_PER_TILE
    for k in range(N_PER_TILE // ZR):
        pltpu.sync_copy(zbuf_v, accum_sh.at[pl.ds(r0 + k * ZR, ZR)])
    rem = N_PER_TILE % ZR
    pltpu.sync_copy(zbuf_v.at[pl.ds(0, rem)],
                    accum_sh.at[pl.ds(r0 + N_PER_TILE - rem, rem)])
    pltpu.sync_copy(ridx3_hbm.at[wid], ix2_v)
    plsc.subcore_barrier()

    e0 = wid * EPW

    def issue_load(ci, b):
        base = e0 + ci * C
        pltpu.async_copy(h3_hbm.at[pl.ds(base, C)], bf[b], slh[b])

    def fire(ci, b):
        pltpu.make_async_copy(h3_hbm.at[pl.ds(0, C)], bf[b], slh[b]).wait()
        pltpu.async_copy(bf[b], accum_sh.at[ix2_v.at[ci]], ss[b], add=True)

    def drain(b):
        pltpu.make_async_copy(h3_hbm.at[pl.ds(0, C)], bf[b], ss[b]).wait()

    for b in range(NBUF):
        issue_load(b, b)

    def body(k, carry):
        g = k * NBUF
        for b in range(NBUF):
            fire(g + b, b)
        for b in range(NBUF):
            drain(b)

            @pl.when(g + NBUF + b < NCH)
            def _():
                issue_load(g + NBUF + b, b)

        return carry

    lax.fori_loop(0, (NCH - 1) // NBUF, body, 0)
    fire(NCH - 1, 0)
    drain(0)
    plsc.subcore_barrier()

    pltpu.sync_copy(accum_sh.at[pl.ds(r0, N_PER_TILE)],
                    out_hbm.at[cid, pl.ds(r0, N_PER_TILE)])


def _segment_sum(h3, ridx3):
    mesh = plsc.VectorSubcoreMesh(core_axis_name="c", subcore_axis_name="s")
    f = pl.kernel(
        lambda h3r, rir, out, *rest: _scatter_body(
            h3r, rir, out,
            rest[0], rest[1:1 + NBUF],
            rest[1 + NBUF], rest[2 + NBUF],
            rest[3 + NBUF:3 + 2 * NBUF],
            rest[3 + 2 * NBUF:3 + 3 * NBUF]),
        out_type=jax.ShapeDtypeStruct((NC, NP, D), jnp.float32),
        mesh=mesh,
        scratch_types=(
            [pltpu.VMEM((NCH, C), jnp.int32)]
            + [pltpu.VMEM((C, D), jnp.float32)] * NBUF
            + [pltpu.VMEM((ZR, D), jnp.float32),
               pltpu.VMEM_SHARED((NP, D), jnp.float32)]
            + [pltpu.SemaphoreType.DMA] * (2 * NBUF)
        ),
    )
    return f(h3, ridx3)


def _sum2_body(pa_ref, pb_ref, o_ref):
    o_ref[...] = (pa_ref[0] + pa_ref[1]) + (pb_ref[0] + pb_ref[1])


def _sum_partials(pa, pb):
    BN = 632
    grid = (NP // BN,)
    return pl.pallas_call(
        _sum2_body,
        grid=grid,
        in_specs=[pl.BlockSpec((2, BN, D), lambda i: (0, i, 0)),
                  pl.BlockSpec((2, BN, D), lambda i: (0, i, 0))],
        out_specs=pl.BlockSpec((BN, D), lambda i: (i, 0)),
        out_shape=jax.ShapeDtypeStruct((NP, D), jnp.float32),
    )(pa, pb)


def kernel(x, edge_idx, W1, b1, W2, b2, W3, b3):
    ridx = edge_idx[0].astype(jnp.int32)
    cidxp = edge_idx[1].astype(jnp.int32) + N_NODES

    xab = _preproj(x, W1, b1).reshape(2 * N_NODES, D)
    r0 = lax.dynamic_slice_in_dim(ridx, 0, EH)
    c0 = lax.dynamic_slice_in_dim(cidxp, 0, EH)
    r1 = lax.dynamic_slice_in_dim(ridx, EH, EH)
    c1 = lax.dynamic_slice_in_dim(cidxp, EH, EH)

    hpre0, ptab = _gather_pairs_pack(xab, r0, c0)
    h3_0 = _mlp_tail(hpre0, W2, b2, W3, b3)
    hpre1 = _gather_pairs_reuse(ptab, r1, c1)
    p0 = _segment_sum(h3_0, r0.reshape(NW, NCH, C))
    h3_1 = _mlp_tail(hpre1, W2, b2, W3, b3)
    p1 = _segment_sum(h3_1, r1.reshape(NW, NCH, C))
    return _sum_partials(p0, p1)[:N_NODES]
